# baseline (speedup 1.0000x reference)
import functools
import math

import jax
import jax.numpy as jnp
from jax import lax
from jax.experimental import pallas as pl
from jax.experimental.pallas import tpu as pltpu
from jax.experimental.pallas import tpu_sc as plsc

VOCAB = 1000000
D = 64
SCALE = math.sqrt(D)
L = 16


def _make_kernel(B, C):
    info = plsc.get_sparse_core_info()
    NC, NS = info.num_cores, info.num_subcores
    NW = NC * NS
    assert B % NW == 0
    b_per_w = B // NW
    assert b_per_w % C == 0
    n_chunks = b_per_w // C

    mesh = plsc.VectorSubcoreMesh(core_axis_name="c", subcore_axis_name="s")

    @functools.partial(
        pl.kernel,
        mesh=mesh,
        out_type=jax.ShapeDtypeStruct((B, D), jnp.float32),
        scratch_types=[
            pltpu.VMEM((C,), jnp.int32),
            pltpu.VMEM((C, D), jnp.float32),
            pltpu.SemaphoreType.DMA,
        ],
        compiler_params=pltpu.CompilerParams(use_tc_tiling_on_sc=False),
    )
    def k(x_hbm, table_hbm, out_hbm, idx_v, rows_v, sem):
        wid = lax.axis_index("s") * NC + lax.axis_index("c")
        base = wid * b_per_w

        def chunk_body(g, carry):
            off = base + g * C
            pltpu.sync_copy(x_hbm.at[pl.ds(off, C)], idx_v)
            pltpu.async_copy(table_hbm.at[idx_v], rows_v, sem).wait()

            def scale_body(r, carry2):
                for rr in range(8):
                    for d in range(D // L):
                        sl = (r * 8 + rr, pl.ds(d * L, L))
                        rows_v[sl] = rows_v[sl] * SCALE
                return carry2

            lax.fori_loop(0, C // 8, scale_body, 0)
            pltpu.sync_copy(rows_v, out_hbm.at[pl.ds(off, C)])
            return carry

        lax.fori_loop(0, n_chunks, chunk_body, 0)

    return k


def kernel(x, table):
    B = x.shape[0] * x.shape[1]
    x_flat = x.reshape(B)
    out = _make_kernel(B, 800)(x_flat, table)
    return out.reshape(x.shape[0], x.shape[1], D)


# speedup vs baseline: 1.0663x; 1.0663x over previous
import functools
import math

import jax
import jax.numpy as jnp
from jax import lax
from jax.experimental import pallas as pl
from jax.experimental.pallas import tpu as pltpu
from jax.experimental.pallas import tpu_sc as plsc

VOCAB = 1000000
D = 64
SCALE = math.sqrt(D)
L = 16


def _make_kernel(B, C):
    info = plsc.get_sparse_core_info()
    NC, NS = info.num_cores, info.num_subcores
    NW = NC * NS
    assert B % NW == 0
    b_per_w = B // NW
    assert b_per_w % C == 0
    n_chunks = b_per_w // C

    mesh = plsc.VectorSubcoreMesh(core_axis_name="c", subcore_axis_name="s")

    @functools.partial(
        pl.kernel,
        mesh=mesh,
        out_type=jax.ShapeDtypeStruct((B, D), jnp.float32),
        scratch_types=[
            pltpu.VMEM((2, C), jnp.int32),
            pltpu.VMEM((2, C, D), jnp.float32),
            [pltpu.SemaphoreType.DMA] * 2,
            [pltpu.SemaphoreType.DMA] * 2,
        ],
        compiler_params=pltpu.CompilerParams(use_tc_tiling_on_sc=False),
    )
    def k(x_hbm, table_hbm, out_hbm, idx_v, rows_v, gsem, wsem):
        wid = lax.axis_index("s") * NC + lax.axis_index("c")
        base = wid * b_per_w

        def scale_chunk(b):
            def scale_body(r, carry):
                for rr in range(8):
                    for d in range(D // L):
                        sl = (b, r * 8 + rr, pl.ds(d * L, L))
                        rows_v[sl] = rows_v[sl] * SCALE
                return carry

            lax.fori_loop(0, C // 8, scale_body, 0)

        for g in range(n_chunks + 1):
            if g < n_chunks:
                b = g % 2
                off = base + g * C
                if g >= 2:
                    pltpu.make_async_copy(
                        rows_v.at[b], out_hbm.at[pl.ds(base + (g - 2) * C, C)],
                        wsem[b]).wait()
                pltpu.sync_copy(x_hbm.at[pl.ds(off, C)], idx_v.at[b])
                pltpu.async_copy(table_hbm.at[idx_v.at[b]], rows_v.at[b],
                                 gsem[b])
            if g >= 1:
                p = (g - 1) % 2
                poff = base + (g - 1) * C
                pltpu.make_async_copy(table_hbm.at[idx_v.at[p]],
                                      rows_v.at[p], gsem[p]).wait()
                scale_chunk(p)
                pltpu.async_copy(rows_v.at[p], out_hbm.at[pl.ds(poff, C)],
                                 wsem[p])

        for g in (n_chunks - 2, n_chunks - 1):
            b = g % 2
            pltpu.make_async_copy(
                rows_v.at[b], out_hbm.at[pl.ds(base + g * C, C)],
                wsem[b]).wait()

    return k


def kernel(x, table):
    B = x.shape[0] * x.shape[1]
    x_flat = x.reshape(B)
    out = _make_kernel(B, 800)(x_flat, table)
    return out.reshape(x.shape[0], x.shape[1], D)
